# Initial kernel scaffold; baseline (speedup 1.0000x reference)
import jax
import jax.numpy as jnp
from jax.experimental import pallas as pl


def kernel(x, emb_q, emb_r):
    raise NotImplementedError("write your pallas kernel here")


# speedup vs baseline: 3.2868x; 3.2868x over previous
import functools

import jax
import jax.numpy as jnp
from jax import lax
from jax.experimental import pallas as pl
from jax.experimental.pallas import tpu as pltpu
from jax.experimental.pallas import tpu_sc as plsc

DIV = 1000
EMB_DIM = 128
L = 16

_info = plsc.get_sparse_core_info()
NC, NS = _info.num_cores, _info.num_subcores
NW = NC * NS

IDX_C = 256
ROW_C = 16


def _pack_table(emb):
    eb = emb.astype(jnp.bfloat16)
    u = lax.bitcast_convert_type(eb, jnp.uint16).astype(jnp.uint32)
    u4 = u.reshape(emb.shape[0], 4, 2, L)
    w = u4[:, :, 0, :] | (u4[:, :, 1, :] << 16)
    return lax.bitcast_convert_type(w, jnp.int32).reshape(emb.shape[0] * 64)


def _take16(v, idx):
    dnums = lax.GatherDimensionNumbers(
        offset_dims=(), collapsed_slice_dims=(0,), start_index_map=(0,))
    return lax.gather(v, idx[:, None], dnums, (1,),
                      mode=lax.GatherScatterMode.PROMISE_IN_BOUNDS)


def _rsqrt_f32(s):
    ib = lax.bitcast_convert_type(s, jnp.int32)
    yi = jnp.int32(0x5F3759DF) - lax.shift_right_logical(ib, 1)
    y = lax.bitcast_convert_type(yi, jnp.float32)
    for _ in range(3):
        y = y * (1.5 - 0.5 * s * y * y)
    return y


@functools.partial(jax.jit, static_argnames=("n_total",))
def _qr_embed_sc(x_flat, qtab, rtab, *, n_total):
    rows_per_w = n_total // NW
    n_blocks = rows_per_w // IDX_C
    chunks_per_block = IDX_C // ROW_C
    mesh = plsc.VectorSubcoreMesh(core_axis_name="c", subcore_axis_name="s")
    mask_hi = jnp.int32(-65536)

    @functools.partial(
        pl.kernel,
        out_type=jax.ShapeDtypeStruct((n_total, EMB_DIM), jnp.float32),
        mesh=mesh,
        scratch_types=[
            pltpu.VMEM((DIV * 64,), jnp.int32),
            pltpu.VMEM((DIV * 64,), jnp.int32),
            pltpu.VMEM((IDX_C,), jnp.int32),
            pltpu.VMEM((IDX_C,), jnp.int32),
            pltpu.VMEM((ROW_C, EMB_DIM), jnp.float32),
        ],
    )
    def body(x_hbm, qtab_hbm, rtab_hbm, out_hbm,
             qtab_v, rtab_v, idxq_v, idxr_v, out_v):
        wid = lax.axis_index("s") * NC + lax.axis_index("c")
        pltpu.sync_copy(qtab_hbm, qtab_v)
        pltpu.sync_copy(rtab_hbm, rtab_v)
        lane = lax.broadcasted_iota(jnp.int32, (L,), 0)
        perms = [(lane + (1 << p)) & jnp.int32(L - 1) for p in range(4)]

        def block_body(bi, carry):
            bbase = wid * rows_per_w + bi * IDX_C
            pltpu.sync_copy(x_hbm.at[pl.ds(bbase, IDX_C)], idxq_v)

            def qr_body(j, carry2):
                v = idxq_v[pl.ds(j * L, L)]
                qv = lax.div(v, jnp.int32(DIV))
                idxq_v[pl.ds(j * L, L)] = qv
                idxr_v[pl.ds(j * L, L)] = v - qv * jnp.int32(DIV)
                return carry2

            lax.fori_loop(0, IDX_C // L, qr_body, 0)

            def chunk_body(ci, carry3):
                qvec = idxq_v[pl.ds(ci * ROW_C, ROW_C)]
                rvec = idxr_v[pl.ds(ci * ROW_C, ROW_C)]
                for k in range(ROW_C):
                    qw = qvec[k] * 64
                    rw = rvec[k] * 64
                    acc = jnp.zeros((L,), jnp.float32)
                    svals = []
                    for b in range(4):
                        wq = qtab_v[pl.ds(qw + b * L, L)]
                        wr = rtab_v[pl.ds(rw + b * L, L)]
                        lo = (lax.bitcast_convert_type(
                                  lax.shift_left(wq, 16), jnp.float32)
                              + lax.bitcast_convert_type(
                                  lax.shift_left(wr, 16), jnp.float32))
                        hi = (lax.bitcast_convert_type(
                                  wq & mask_hi, jnp.float32)
                              + lax.bitcast_convert_type(
                                  wr & mask_hi, jnp.float32))
                        acc = acc + lo * lo + hi * hi
                        svals += [lo, hi]
                    for pv in perms:
                        acc = acc + _take16(acc, pv)
                    inv = _rsqrt_f32(jnp.maximum(acc, jnp.float32(1e-24)))
                    for t, s in enumerate(svals):
                        out_v[k, pl.ds(t * L, L)] = s * inv
                pltpu.sync_copy(
                    out_v, out_hbm.at[pl.ds(bbase + ci * ROW_C, ROW_C)])
                return carry3

            lax.fori_loop(0, chunks_per_block, chunk_body, 0)
            return carry

        lax.fori_loop(0, n_blocks, block_body, 0)

    return body(x_flat, qtab, rtab)


def kernel(x, emb_q, emb_r):
    n_total = x.shape[0] * x.shape[1]
    x_flat = x.reshape(n_total)
    out = _qr_embed_sc(x_flat, _pack_table(emb_q), _pack_table(emb_r),
                       n_total=n_total)
    return out.reshape(x.shape[0], x.shape[1], EMB_DIM)


# speedup vs baseline: 3.7751x; 1.1486x over previous
import functools

import jax
import jax.numpy as jnp
from jax import lax
from jax.experimental import pallas as pl
from jax.experimental.pallas import tpu as pltpu
from jax.experimental.pallas import tpu_sc as plsc

DIV = 1000
EMB_DIM = 128
L = 16

_info = plsc.get_sparse_core_info()
NC, NS = _info.num_cores, _info.num_subcores
NW = NC * NS

IDX_C = 256
ROW_C = 8


def _pack_table(emb):
    eb = emb.astype(jnp.bfloat16)
    u = lax.bitcast_convert_type(eb, jnp.uint16).astype(jnp.uint32)
    u4 = u.reshape(emb.shape[0], 4, 2, L)
    w = u4[:, :, 0, :] | (u4[:, :, 1, :] << 16)
    return lax.bitcast_convert_type(w, jnp.int32).reshape(emb.shape[0] * 64)


def _take16(v, idx):
    dnums = lax.GatherDimensionNumbers(
        offset_dims=(), collapsed_slice_dims=(0,), start_index_map=(0,))
    return lax.gather(v, idx[:, None], dnums, (1,),
                      mode=lax.GatherScatterMode.PROMISE_IN_BOUNDS)


def _rsqrt_f32(s):
    ib = lax.bitcast_convert_type(s, jnp.int32)
    yi = jnp.int32(0x5F3759DF) - lax.shift_right_logical(ib, 1)
    y = lax.bitcast_convert_type(yi, jnp.float32)
    for _ in range(2):
        y = y * (1.5 - 0.5 * s * y * y)
    return y


@functools.partial(jax.jit, static_argnames=("n_total",))
def _qr_embed_sc(x_flat, qtab, rtab, *, n_total):
    rows_per_w = n_total // NW
    n_blocks = rows_per_w // IDX_C
    chunks_per_block = IDX_C // ROW_C
    mesh = plsc.VectorSubcoreMesh(core_axis_name="c", subcore_axis_name="s")
    mask_hi = jnp.int32(-65536)

    @functools.partial(
        pl.kernel,
        out_type=jax.ShapeDtypeStruct((n_total, EMB_DIM), jnp.float32),
        mesh=mesh,
        scratch_types=[
            pltpu.VMEM((DIV * 64,), jnp.int32),
            pltpu.VMEM((DIV * 64,), jnp.int32),
            pltpu.VMEM((IDX_C,), jnp.int32),
            pltpu.VMEM((IDX_C,), jnp.int32),
            pltpu.VMEM((ROW_C, EMB_DIM), jnp.float32),
            pltpu.VMEM((ROW_C, EMB_DIM), jnp.float32),
            pltpu.SemaphoreType.DMA,
            pltpu.SemaphoreType.DMA,
        ],
    )
    def body(x_hbm, qtab_hbm, rtab_hbm, out_hbm,
             qtab_v, rtab_v, idxq_v, idxr_v, out_a, out_b, sem_a, sem_b):
        wid = lax.axis_index("s") * NC + lax.axis_index("c")
        pltpu.sync_copy(qtab_hbm, qtab_v)
        pltpu.sync_copy(rtab_hbm, rtab_v)
        lane = lax.broadcasted_iota(jnp.int32, (L,), 0)
        perms = [(lane + (1 << p)) & jnp.int32(L - 1) for p in range(4)]

        def block_body(bi, carry):
            bbase = wid * rows_per_w + bi * IDX_C
            pltpu.sync_copy(x_hbm.at[pl.ds(bbase, IDX_C)], idxq_v)

            def qr_body(j, carry2):
                v = idxq_v[pl.ds(j * L, L)]
                qv = lax.div(v, jnp.int32(DIV))
                idxq_v[pl.ds(j * L, L)] = qv
                idxr_v[pl.ds(j * L, L)] = v - qv * jnp.int32(DIV)
                return carry2

            lax.fori_loop(0, IDX_C // L, qr_body, 0)

            def group_body(g, carry3):
                qvec = idxq_v[pl.ds(g * L, L)]
                rvec = idxr_v[pl.ds(g * L, L)]
                glob = bi * (IDX_C // L) + g
                for half, (buf, sem) in enumerate(
                        ((out_a, sem_a), (out_b, sem_b))):
                    dst = out_hbm.at[
                        pl.ds(bbase + g * L + half * ROW_C, ROW_C)]

                    @pl.when(glob > 0)
                    def _wait_prev():
                        pltpu.make_async_copy(buf, dst, sem).wait()

                    for k in range(ROW_C):
                        qw = qvec[half * ROW_C + k] * 64
                        rw = rvec[half * ROW_C + k] * 64
                        acc = jnp.zeros((L,), jnp.float32)
                        svals = []
                        for b in range(4):
                            wq = qtab_v[pl.ds(qw + b * L, L)]
                            wr = rtab_v[pl.ds(rw + b * L, L)]
                            lo = (lax.bitcast_convert_type(
                                      lax.shift_left(wq, 16), jnp.float32)
                                  + lax.bitcast_convert_type(
                                      lax.shift_left(wr, 16), jnp.float32))
                            hi = (lax.bitcast_convert_type(
                                      wq & mask_hi, jnp.float32)
                                  + lax.bitcast_convert_type(
                                      wr & mask_hi, jnp.float32))
                            acc = acc + lo * lo + hi * hi
                            svals += [lo, hi]
                        for pv in perms:
                            acc = acc + _take16(acc, pv)
                        inv = _rsqrt_f32(
                            jnp.maximum(acc, jnp.float32(1e-24)))
                        for t, s in enumerate(svals):
                            buf[k, pl.ds(t * L, L)] = s * inv
                    pltpu.async_copy(buf, dst, sem)
                return carry3

            lax.fori_loop(0, IDX_C // L, group_body, 0)
            return carry

        lax.fori_loop(0, n_blocks, block_body, 0)
        tail = out_hbm.at[pl.ds(wid * rows_per_w, ROW_C)]
        pltpu.make_async_copy(out_a, tail, sem_a).wait()
        pltpu.make_async_copy(out_b, tail, sem_b).wait()

    return body(x_flat, qtab, rtab)


def kernel(x, emb_q, emb_r):
    n_total = x.shape[0] * x.shape[1]
    x_flat = x.reshape(n_total)
    out = _qr_embed_sc(x_flat, _pack_table(emb_q), _pack_table(emb_r),
                       n_total=n_total)
    return out.reshape(x.shape[0], x.shape[1], EMB_DIM)
